# Initial kernel scaffold; baseline (speedup 1.0000x reference)
import jax
import jax.numpy as jnp
from jax.experimental import pallas as pl


def kernel(edge_attr, table0, table1, table2, gamma, beta):
    raise NotImplementedError("write your pallas kernel here")


# speedup vs baseline: 1.6421x; 1.6421x over previous
import functools

import jax
import jax.numpy as jnp
from jax import lax
from jax.experimental import pallas as pl
from jax.experimental.pallas import tpu as pltpu
from jax.experimental.pallas import tpu_sc as plsc

_NC = 2
_NS = 16
_NW = _NC * _NS
_CPAD = 128


def _idx_hist_body(r1r2, r2, r0m1, r1m1, r2m1, blk, i0_ref, i1_ref, i2_ref,
                   c_ref, counts_ref):
    j0 = jnp.clip(i0_ref[...], 0, r0m1)
    j1 = jnp.clip(i1_ref[...], 0, r1m1)
    j2 = jnp.clip(i2_ref[...], 0, r2m1)
    c = j0 * r1r2 + j1 * r2 + j2
    c_ref[...] = c
    lanes = lax.broadcasted_iota(jnp.int32, (blk, _CPAD), 1)
    onehot = (jnp.broadcast_to(c, (blk, _CPAD)) == lanes).astype(jnp.float32)

    @pl.when(pl.program_id(0) == 0)
    def _():
        counts_ref[...] = jnp.zeros_like(counts_ref)

    counts_ref[...] += jnp.sum(onehot, axis=0, keepdims=True)


def _stats_body(r0, r1, r2, n_edges, dim, counts_ref, t0_ref, t1_ref, t2_ref,
                g_ref, b_ref, tn_ref):
    r = lax.broadcasted_iota(jnp.int32, (_CPAD, 1), 0)
    a0 = r // (r1 * r2)
    a1 = (r // r2) % r1
    a2 = r % r2
    t = jnp.zeros((_CPAD, dim), jnp.float32)
    for j in range(r0):
        t = t + jnp.where(a0 == j, 1.0, 0.0) * t0_ref[j:j + 1, :]
    for j in range(r1):
        t = t + jnp.where(a1 == j, 1.0, 0.0) * t1_ref[j:j + 1, :]
    for j in range(r2):
        t = t + jnp.where(a2 == j, 1.0, 0.0) * t2_ref[j:j + 1, :]
    cnt = counts_ref[...]
    inv_n = 1.0 / n_edges
    mean = jnp.dot(cnt, t, preferred_element_type=jnp.float32) * inv_n
    tc = t - mean
    var = jnp.dot(cnt, tc * tc, preferred_element_type=jnp.float32) * inv_n
    scale = g_ref[...] * lax.rsqrt(var + 1e-5)
    tn_ref[...] = tc * scale + b_ref[...]


def _expand_body(nch, ch, dim, c_hbm, tn_hbm, out_hbm, idx_v, rows0, rows1,
                 g0, g1, s0, s1):
    wid = lax.axis_index("s") * _NC + lax.axis_index("c")
    row_base = wid * (nch * ch)
    pltpu.sync_copy(c_hbm.at[wid], idx_v)
    rows = (rows0, rows1)
    gsems = (g0, g1)
    ssems = (s0, s1)
    pltpu.async_copy(tn_hbm.at[idx_v.at[0]], rows0, g0)
    pltpu.async_copy(tn_hbm.at[idx_v.at[1]], rows1, g1)

    def step(g, carry):
        for b in range(2):
            j = g * 2 + b
            pltpu.make_async_copy(
                tn_hbm.at[idx_v.at[j]], rows[b], gsems[b]).wait()
            row0 = pl.multiple_of(row_base + j * ch, 8)
            pltpu.async_copy(
                rows[b], out_hbm.at[pl.ds(row0, ch)], ssems[b]).wait()

            @pl.when(j + 2 < nch)
            def _():
                pltpu.async_copy(
                    tn_hbm.at[idx_v.at[j + 2]], rows[b], gsems[b])
        return carry

    lax.fori_loop(0, nch // 2, step, 0)


def kernel(edge_attr, table0, table1, table2, gamma, beta):
    n_edges, _ = edge_attr.shape
    r0, dim = table0.shape
    r1 = table1.shape[0]
    r2 = table2.shape[0]

    ea = edge_attr.astype(jnp.int32)
    i0 = ea[:, 0:1]
    i1 = ea[:, 1:2]
    i2 = ea[:, 2:3]

    blk = 3200
    grid = n_edges // blk
    c_col, counts = pl.pallas_call(
        functools.partial(_idx_hist_body, r1 * r2, r2, r0 - 1, r1 - 1, r2 - 1,
                          blk),
        grid=(grid,),
        in_specs=[pl.BlockSpec((blk, 1), lambda i: (i, 0))] * 3,
        out_specs=[pl.BlockSpec((blk, 1), lambda i: (i, 0)),
                   pl.BlockSpec((1, _CPAD), lambda i: (0, 0))],
        out_shape=[jax.ShapeDtypeStruct((n_edges, 1), jnp.int32),
                   jax.ShapeDtypeStruct((1, _CPAD), jnp.float32)],
    )(i0, i1, i2)

    tn = pl.pallas_call(
        functools.partial(_stats_body, r0, r1, r2, float(n_edges), dim),
        out_shape=jax.ShapeDtypeStruct((_CPAD, dim), jnp.float32),
    )(counts, table0, table1, table2, gamma.reshape(1, dim),
      beta.reshape(1, dim))

    ch = 40
    rows_w = n_edges // _NW
    nch = rows_w // ch
    c3d = c_col.reshape(_NW, nch, ch)

    mesh = plsc.VectorSubcoreMesh(core_axis_name="c", subcore_axis_name="s")
    expand = functools.partial(
        pl.kernel,
        mesh=mesh,
        out_type=jax.ShapeDtypeStruct((n_edges, dim), jnp.float32),
        scratch_types=[
            pltpu.VMEM((nch, ch), jnp.int32),
            pltpu.VMEM((ch, dim), jnp.float32),
            pltpu.VMEM((ch, dim), jnp.float32),
            pltpu.SemaphoreType.DMA,
            pltpu.SemaphoreType.DMA,
            pltpu.SemaphoreType.DMA,
            pltpu.SemaphoreType.DMA,
        ],
    )(functools.partial(_expand_body, nch, ch, dim))
    return expand(c3d, tn)


# speedup vs baseline: 1.8111x; 1.1029x over previous
import functools

import jax
import jax.numpy as jnp
from jax import lax
from jax.experimental import pallas as pl
from jax.experimental.pallas import tpu as pltpu
from jax.experimental.pallas import tpu_sc as plsc

_NC = 2
_NS = 16
_NW = _NC * _NS
_CPAD = 128
_L = 16


def _idx_hist_body(hch, r1r2, r2, r0m1, r1m1, r2m1, ea_hbm, c_hbm, counts_hbm,
                   i0v, i1v, i2v, cv, idx2v, ones_v, stage_v, hist_vm,
                   shared_h, sem0, sem1, sem2, ssem):
    cid = lax.axis_index("c")
    sid = lax.axis_index("s")
    wid = sid * _NC + cid
    base = wid * hch
    cp0 = pltpu.async_copy(ea_hbm.at[0, 0, pl.ds(base, hch)], i0v.at[0], sem0)
    cp1 = pltpu.async_copy(ea_hbm.at[1, 0, pl.ds(base, hch)], i1v.at[0], sem1)
    cp2 = pltpu.async_copy(ea_hbm.at[2, 0, pl.ds(base, hch)], i2v.at[0], sem2)

    for k in range(_CPAD // _L):
        ones_v[0, pl.ds(k * _L, _L)] = jnp.full((_L,), 1.0, jnp.float32)
        stage_v[0, pl.ds(k * _L, _L)] = jnp.zeros((_L,), jnp.float32)

    def zero_step(k, carry):
        hist_vm[pl.ds(k * _L, _L)] = jnp.zeros((_L,), jnp.float32)
        return carry

    lax.fori_loop(0, (_CPAD * _CPAD) // _L, zero_step, 0)

    @pl.when(sid == 0)
    def _():
        pltpu.sync_copy(hist_vm, shared_h)

    plsc.subcore_barrier()
    cp0.wait()
    cp1.wait()
    cp2.wait()

    def step(k, carry):
        lane = lax.iota(jnp.int32, _L)
        off = k * _L
        j0 = jnp.clip(i0v[0, pl.ds(off, _L)], 0, r0m1)
        j1 = jnp.clip(i1v[0, pl.ds(off, _L)], 0, r1m1)
        j2 = jnp.clip(i2v[0, pl.ds(off, _L)], 0, r2m1)
        c = j0 * r1r2 + j1 * r2 + j2
        cv[0, pl.ds(off, _L)] = c
        pos = (k % (_CPAD // _L)) * _L + lane
        idx2v[0, pl.ds(off, _L)] = pos * _CPAD + c
        return carry

    lax.fori_loop(0, hch // _L, step, 0)

    def fire(j, carry):
        joff = pl.multiple_of(j * _CPAD, _CPAD)
        pltpu.async_copy(ones_v.at[0],
                         shared_h.at[idx2v.at[0, pl.ds(joff, _CPAD)]],
                         ssem, add=True)
        return carry

    lax.fori_loop(0, hch // _CPAD, fire, 0)

    def drain(j, carry):
        joff = pl.multiple_of(j * _CPAD, _CPAD)
        pltpu.make_async_copy(ones_v.at[0],
                              shared_h.at[idx2v.at[0, pl.ds(joff, _CPAD)]],
                              ssem).wait()
        return carry

    lax.fori_loop(0, hch // _CPAD, drain, 0)
    pltpu.sync_copy(cv, c_hbm.at[wid])
    plsc.subcore_barrier()

    @pl.when(sid == 0)
    def _():
        pltpu.sync_copy(shared_h, hist_vm)

        def red_step(p, carry):
            poff = p * _CPAD
            for k8 in range(_CPAD // _L):
                o = k8 * _L
                stage_v[0, pl.ds(o, _L)] = (
                    stage_v[0, pl.ds(o, _L)] + hist_vm[pl.ds(poff + o, _L)])
            return carry

        lax.fori_loop(0, _CPAD, red_step, 0)
        pltpu.sync_copy(stage_v, counts_hbm.at[cid])


def _stats_body(r0, r1, r2, n_edges, n_pad, dim, counts_ref, t0_ref, t1_ref,
                t2_ref, g_ref, b_ref, tn_ref):
    r = lax.broadcasted_iota(jnp.int32, (_CPAD, 1), 0)
    a0 = r // (r1 * r2)
    a1 = (r // r2) % r1
    a2 = r % r2
    t = jnp.zeros((_CPAD, dim), jnp.float32)
    for j in range(r0):
        t = t + jnp.where(a0 == j, 1.0, 0.0) * t0_ref[j:j + 1, :]
    for j in range(r1):
        t = t + jnp.where(a1 == j, 1.0, 0.0) * t1_ref[j:j + 1, :]
    for j in range(r2):
        t = t + jnp.where(a2 == j, 1.0, 0.0) * t2_ref[j:j + 1, :]
    cnt = jnp.sum(counts_ref[...], axis=0, keepdims=True)
    bin_iota = lax.broadcasted_iota(jnp.int32, (1, _CPAD), 1)
    cnt = cnt - jnp.where(bin_iota == 0, n_pad, 0.0)
    inv_n = 1.0 / n_edges
    mean = jnp.dot(cnt, t, preferred_element_type=jnp.float32,
                   precision=lax.Precision.HIGHEST) * inv_n
    tc = t - mean
    var = jnp.dot(cnt, tc * tc, preferred_element_type=jnp.float32,
                  precision=lax.Precision.HIGHEST) * inv_n
    scale = g_ref[...] / jnp.sqrt(var + 1e-5)
    tn_ref[...] = tc * scale + b_ref[...]


def _expand_body(hch, ch, dim, full, rem_nch, c_hbm, tn_hbm, out_hbm, idx_v,
                 rows0, rows1, g0, g1, s0, s1):
    wid = lax.axis_index("s") * _NC + lax.axis_index("c")
    row_base = wid * hch
    nch = jnp.where(wid < full, hch // ch, rem_nch)
    pltpu.sync_copy(c_hbm.at[wid], idx_v)
    rows = (rows0, rows1)
    gsems = (g0, g1)
    ssems = (s0, s1)
    pltpu.async_copy(tn_hbm.at[idx_v.at[0, pl.ds(0, ch)]], rows0, g0)
    pltpu.async_copy(tn_hbm.at[idx_v.at[0, pl.ds(ch, ch)]], rows1, g1)

    def step(g, carry):
        for b in range(2):
            j = g * 2 + b
            ioff = pl.multiple_of(j * ch, _CPAD)
            pltpu.make_async_copy(
                tn_hbm.at[idx_v.at[0, pl.ds(ioff, ch)]], rows[b],
                gsems[b]).wait()
            row0 = pl.multiple_of(row_base + j * ch, 8)
            pltpu.async_copy(
                rows[b], out_hbm.at[pl.ds(row0, ch)], ssems[b]).wait()

            @pl.when(j + 2 < nch)
            def _():
                ioff2 = pl.multiple_of((j + 2) * ch, _CPAD)
                pltpu.async_copy(
                    tn_hbm.at[idx_v.at[0, pl.ds(ioff2, ch)]], rows[b],
                    gsems[b])
        return carry

    lax.fori_loop(0, nch // 2, step, 0)


def kernel(edge_attr, table0, table1, table2, gamma, beta):
    n_edges, _ = edge_attr.shape
    r0, dim = table0.shape
    r1 = table1.shape[0]
    r2 = table2.shape[0]

    hch = ((n_edges + _NW * _CPAD - 1) // (_NW * _CPAD)) * _CPAD
    n_pad = _NW * hch - n_edges
    ea = edge_attr.astype(jnp.int32)
    ea = jnp.concatenate([ea, jnp.zeros((n_pad, 3), jnp.int32)], axis=0)
    ea_t = ea.T.reshape(3, 1, _NW * hch)

    mesh = plsc.VectorSubcoreMesh(core_axis_name="c", subcore_axis_name="s")

    idx_hist = functools.partial(
        pl.kernel,
        mesh=mesh,
        out_type=[jax.ShapeDtypeStruct((_NW, 1, hch), jnp.int32),
                  jax.ShapeDtypeStruct((_NC, 1, _CPAD), jnp.float32)],
        scratch_types=[
            pltpu.VMEM((1, hch), jnp.int32),
            pltpu.VMEM((1, hch), jnp.int32),
            pltpu.VMEM((1, hch), jnp.int32),
            pltpu.VMEM((1, hch), jnp.int32),
            pltpu.VMEM((1, hch), jnp.int32),
            pltpu.VMEM((1, _CPAD), jnp.float32),
            pltpu.VMEM((1, _CPAD), jnp.float32),
            pltpu.VMEM((_CPAD * _CPAD,), jnp.float32),
            pltpu.VMEM_SHARED((_CPAD * _CPAD,), jnp.float32),
            pltpu.SemaphoreType.DMA,
            pltpu.SemaphoreType.DMA,
            pltpu.SemaphoreType.DMA,
            pltpu.SemaphoreType.DMA,
        ],
    )(functools.partial(_idx_hist_body, hch, r1 * r2, r2, r0 - 1, r1 - 1,
                        r2 - 1))
    c3, counts = idx_hist(ea_t)

    tn = pl.pallas_call(
        functools.partial(_stats_body, r0, r1, r2, float(n_edges),
                          float(n_pad), dim),
        out_shape=jax.ShapeDtypeStruct((_CPAD, dim), jnp.float32),
    )(counts.reshape(_NC, _CPAD), table0, table1, table2,
      gamma.reshape(1, dim), beta.reshape(1, dim))

    ch = _CPAD
    full = n_edges // hch
    rem_nch = (n_edges - full * hch) // ch

    expand = functools.partial(
        pl.kernel,
        mesh=mesh,
        out_type=jax.ShapeDtypeStruct((n_edges, dim), jnp.float32),
        scratch_types=[
            pltpu.VMEM((1, hch), jnp.int32),
            pltpu.VMEM((ch, dim), jnp.float32),
            pltpu.VMEM((ch, dim), jnp.float32),
            pltpu.SemaphoreType.DMA,
            pltpu.SemaphoreType.DMA,
            pltpu.SemaphoreType.DMA,
            pltpu.SemaphoreType.DMA,
        ],
    )(functools.partial(_expand_body, hch, ch, dim, full, rem_nch))
    return expand(c3, tn)


# speedup vs baseline: 4.0058x; 2.2118x over previous
import functools

import jax
import jax.numpy as jnp
from jax import lax
from jax.experimental import pallas as pl
from jax.experimental.pallas import tpu as pltpu
from jax.experimental.pallas import tpu_sc as plsc

_NC = 2
_NS = 16
_NW = _NC * _NS
_CPAD = 128
_L = 16


def _idx_hist_body(hch, r1r2, r2, r0m1, r1m1, r2m1, ea_hbm, c_hbm, counts_hbm,
                   i0v, i1v, i2v, cv, idx2v, ones_v, stage_v, hist_vm,
                   shared_h, sem0, sem1, sem2, ssem):
    cid = lax.axis_index("c")
    sid = lax.axis_index("s")
    wid = sid * _NC + cid
    base = wid * hch
    cp0 = pltpu.async_copy(ea_hbm.at[0, 0, pl.ds(base, hch)], i0v.at[0], sem0)
    cp1 = pltpu.async_copy(ea_hbm.at[1, 0, pl.ds(base, hch)], i1v.at[0], sem1)
    cp2 = pltpu.async_copy(ea_hbm.at[2, 0, pl.ds(base, hch)], i2v.at[0], sem2)

    for k in range(_CPAD // _L):
        ones_v[0, pl.ds(k * _L, _L)] = jnp.full((_L,), 1.0, jnp.float32)
        stage_v[0, pl.ds(k * _L, _L)] = jnp.zeros((_L,), jnp.float32)

    def zero_step(k, carry):
        hist_vm[pl.ds(k * _L, _L)] = jnp.zeros((_L,), jnp.float32)
        return carry

    lax.fori_loop(0, (_CPAD * _CPAD) // _L, zero_step, 0)

    @pl.when(sid == 0)
    def _():
        pltpu.sync_copy(hist_vm, shared_h)

    plsc.subcore_barrier()
    cp0.wait()
    cp1.wait()
    cp2.wait()

    def step(k, carry):
        lane = lax.iota(jnp.int32, _L)
        off = k * _L
        j0 = jnp.clip(i0v[0, pl.ds(off, _L)], 0, r0m1)
        j1 = jnp.clip(i1v[0, pl.ds(off, _L)], 0, r1m1)
        j2 = jnp.clip(i2v[0, pl.ds(off, _L)], 0, r2m1)
        c = j0 * r1r2 + j1 * r2 + j2
        cv[0, pl.ds(off, _L)] = c
        pos = (k % (_CPAD // _L)) * _L + lane
        idx2v[0, pl.ds(off, _L)] = pos * _CPAD + c
        return carry

    lax.fori_loop(0, hch // _L, step, 0)

    def fire(j, carry):
        joff = pl.multiple_of(j * _CPAD, _CPAD)
        pltpu.async_copy(ones_v.at[0],
                         shared_h.at[idx2v.at[0, pl.ds(joff, _CPAD)]],
                         ssem, add=True)
        return carry

    lax.fori_loop(0, hch // _CPAD, fire, 0)

    def drain(j, carry):
        joff = pl.multiple_of(j * _CPAD, _CPAD)
        pltpu.make_async_copy(ones_v.at[0],
                              shared_h.at[idx2v.at[0, pl.ds(joff, _CPAD)]],
                              ssem).wait()
        return carry

    lax.fori_loop(0, hch // _CPAD, drain, 0)
    pltpu.sync_copy(cv, c_hbm.at[wid])
    plsc.subcore_barrier()

    @pl.when(sid == 0)
    def _():
        pltpu.sync_copy(shared_h, hist_vm)

        def red_step(p, carry):
            poff = p * _CPAD
            for k8 in range(_CPAD // _L):
                o = k8 * _L
                stage_v[0, pl.ds(o, _L)] = (
                    stage_v[0, pl.ds(o, _L)] + hist_vm[pl.ds(poff + o, _L)])
            return carry

        lax.fori_loop(0, _CPAD, red_step, 0)
        pltpu.sync_copy(stage_v, counts_hbm.at[cid])


def _stats_body(r0, r1, r2, n_edges, n_pad, dim, counts_ref, t0_ref, t1_ref,
                t2_ref, g_ref, b_ref, tn_ref):
    r = lax.broadcasted_iota(jnp.int32, (_CPAD, 1), 0)
    a0 = r // (r1 * r2)
    a1 = (r // r2) % r1
    a2 = r % r2
    t = jnp.zeros((_CPAD, dim), jnp.float32)
    for j in range(r0):
        t = t + jnp.where(a0 == j, 1.0, 0.0) * t0_ref[j:j + 1, :]
    for j in range(r1):
        t = t + jnp.where(a1 == j, 1.0, 0.0) * t1_ref[j:j + 1, :]
    for j in range(r2):
        t = t + jnp.where(a2 == j, 1.0, 0.0) * t2_ref[j:j + 1, :]
    cnt = jnp.sum(counts_ref[...], axis=0, keepdims=True)
    bin_iota = lax.broadcasted_iota(jnp.int32, (1, _CPAD), 1)
    cnt = cnt - jnp.where(bin_iota == 0, n_pad, 0.0)
    inv_n = 1.0 / n_edges
    mean = jnp.dot(cnt, t, preferred_element_type=jnp.float32,
                   precision=lax.Precision.HIGHEST) * inv_n
    tc = t - mean
    var = jnp.dot(cnt, tc * tc, preferred_element_type=jnp.float32,
                  precision=lax.Precision.HIGHEST) * inv_n
    scale = g_ref[...] / jnp.sqrt(var + 1e-5)
    tn_ref[...] = tc * scale + b_ref[...]


def _expand_body(hch, ch, dim, full, rem_nch, c_hbm, tn_hbm, out_hbm, idx_v,
                 tn_vm, stage0, stage1, tsem, s0, s1):
    wid = lax.axis_index("s") * _NC + lax.axis_index("c")
    row_base = wid * hch
    nch = jnp.where(wid < full, hch // ch, rem_nch)
    cpt = pltpu.async_copy(tn_hbm, tn_vm, tsem)
    pltpu.sync_copy(c_hbm.at[wid], idx_v)
    cpt.wait()
    stages = (stage0, stage1)
    ssems = (s0, s1)
    nt = dim // _L

    def step(p, carry):
        for b in range(2):
            j = p * 2 + b

            @pl.when(j >= 2)
            def _():
                row_p = pl.multiple_of(row_base + (j - 2) * ch, 8)
                pltpu.make_async_copy(
                    stages[b], out_hbm.at[pl.ds(row_p, ch)], ssems[b]).wait()

            @plsc.parallel_loop(0, ch // _L, unroll=2)
            def grp(g):
                cvec = idx_v[0, pl.ds(j * ch + g * _L, _L)]
                for l in range(_L):
                    base = cvec[l] * dim
                    row = g * _L + l
                    vals = [tn_vm[0, pl.ds(base + t * _L, _L)]
                            for t in range(nt)]
                    for t in range(nt):
                        stages[b][row, pl.ds(t * _L, _L)] = vals[t]
            row0 = pl.multiple_of(row_base + j * ch, 8)
            pltpu.async_copy(stages[b], out_hbm.at[pl.ds(row0, ch)], ssems[b])
        return carry

    lax.fori_loop(0, nch // 2, step, 0)
    for b in range(2):
        jlast = nch - 2 + b
        row_p = pl.multiple_of(row_base + jlast * ch, 8)
        pltpu.make_async_copy(
            stages[b], out_hbm.at[pl.ds(row_p, ch)], ssems[b]).wait()


def kernel(edge_attr, table0, table1, table2, gamma, beta):
    n_edges, _ = edge_attr.shape
    r0, dim = table0.shape
    r1 = table1.shape[0]
    r2 = table2.shape[0]

    hch = ((n_edges + _NW * _CPAD - 1) // (_NW * _CPAD)) * _CPAD
    n_pad = _NW * hch - n_edges
    ea = edge_attr.astype(jnp.int32)
    ea = jnp.concatenate([ea, jnp.zeros((n_pad, 3), jnp.int32)], axis=0)
    ea_t = ea.T.reshape(3, 1, _NW * hch)

    mesh = plsc.VectorSubcoreMesh(core_axis_name="c", subcore_axis_name="s")

    idx_hist = functools.partial(
        pl.kernel,
        mesh=mesh,
        out_type=[jax.ShapeDtypeStruct((_NW, 1, hch), jnp.int32),
                  jax.ShapeDtypeStruct((_NC, 1, _CPAD), jnp.float32)],
        scratch_types=[
            pltpu.VMEM((1, hch), jnp.int32),
            pltpu.VMEM((1, hch), jnp.int32),
            pltpu.VMEM((1, hch), jnp.int32),
            pltpu.VMEM((1, hch), jnp.int32),
            pltpu.VMEM((1, hch), jnp.int32),
            pltpu.VMEM((1, _CPAD), jnp.float32),
            pltpu.VMEM((1, _CPAD), jnp.float32),
            pltpu.VMEM((_CPAD * _CPAD,), jnp.float32),
            pltpu.VMEM_SHARED((_CPAD * _CPAD,), jnp.float32),
            pltpu.SemaphoreType.DMA,
            pltpu.SemaphoreType.DMA,
            pltpu.SemaphoreType.DMA,
            pltpu.SemaphoreType.DMA,
        ],
    )(functools.partial(_idx_hist_body, hch, r1 * r2, r2, r0 - 1, r1 - 1,
                        r2 - 1))
    c3, counts = idx_hist(ea_t)

    tn = pl.pallas_call(
        functools.partial(_stats_body, r0, r1, r2, float(n_edges),
                          float(n_pad), dim),
        out_shape=jax.ShapeDtypeStruct((_CPAD, dim), jnp.float32),
    )(counts.reshape(_NC, _CPAD), table0, table1, table2,
      gamma.reshape(1, dim), beta.reshape(1, dim))

    ch = 64
    full = n_edges // hch
    rem_nch = (n_edges - full * hch) // ch

    expand = functools.partial(
        pl.kernel,
        mesh=mesh,
        out_type=jax.ShapeDtypeStruct((n_edges, dim), jnp.float32),
        scratch_types=[
            pltpu.VMEM((1, hch), jnp.int32),
            pltpu.VMEM((1, _CPAD * dim), jnp.float32),
            pltpu.VMEM((ch, dim), jnp.float32),
            pltpu.VMEM((ch, dim), jnp.float32),
            pltpu.SemaphoreType.DMA,
            pltpu.SemaphoreType.DMA,
            pltpu.SemaphoreType.DMA,
        ],
    )(functools.partial(_expand_body, hch, ch, dim, full, rem_nch))
    return expand(c3, tn.reshape(1, _CPAD * dim))
